# baseline (speedup 1.0000x reference)
import functools

import jax
import jax.numpy as jnp
import numpy as np
from jax import lax
from jax.experimental import pallas as pl
from jax.experimental.pallas import tpu as pltpu
from jax.experimental.pallas import tpu_sc as plsc

N = 10000
E = 320000
DF = 128
NS = 8
NG = 16
DH = 128

NB = 10
BN = N // NB

NC = 2
NSUB = 16
NW = NC * NSUB
EPW = E // NW
K = 40
NCH = EPW // K
NPAD = 10240
RPT = NPAD // NSUB
DW = DH + 16


def _pre_body(x_ref, bt_ref, sc_ref, W1_ref, b1_ref, A_ref, B_ref):
    xb = x_ref[...]
    bt = bt_ref[0, 0, :]
    oh = (bt[:, None] == lax.broadcasted_iota(jnp.int32, (1, NG), 1)
          ).astype(jnp.float32)
    s = jnp.dot(oh, sc_ref[...], preferred_element_type=jnp.float32)
    w1ax = W1_ref[0:DF, :]
    w1as = W1_ref[DF:DF + NS, :]
    w1bx = W1_ref[DF + NS:2 * DF + NS, :]
    w1bs = W1_ref[2 * DF + NS:, :]
    b1 = b1_ref[0, :]
    A_ref[...] = (jnp.dot(xb, w1ax - w1bx, preferred_element_type=jnp.float32)
                  + jnp.dot(s, w1as - w1bs, preferred_element_type=jnp.float32)
                  + b1[None, :])
    B_ref[...] = (jnp.dot(xb, w1bx, preferred_element_type=jnp.float32)
                  + jnp.dot(s, w1bs, preferred_element_type=jnp.float32))


_pre = pl.pallas_call(
    _pre_body,
    grid=(NB,),
    in_specs=[
        pl.BlockSpec((BN, DF), lambda i: (i, 0)),
        pl.BlockSpec((1, 1, BN), lambda i: (i, 0, 0)),
        pl.BlockSpec((NG, NS), lambda i: (0, 0)),
        pl.BlockSpec((2 * (DF + NS), DH), lambda i: (0, 0)),
        pl.BlockSpec((1, DH), lambda i: (0, 0)),
    ],
    out_specs=[pl.BlockSpec((BN, DH), lambda i: (i, 0)),
               pl.BlockSpec((BN, DH), lambda i: (i, 0))],
    out_shape=[jax.ShapeDtypeStruct((N, DH), jnp.float32),
               jax.ShapeDtypeStruct((N, DH), jnp.float32)],
)


_mesh = plsc.VectorSubcoreMesh(core_axis_name="c", subcore_axis_name="s")


@functools.partial(
    pl.kernel,
    out_type=jax.ShapeDtypeStruct((NC, NPAD, DW), jnp.float32),
    mesh=_mesh,
    scratch_types=[
        pltpu.VMEM((K,), jnp.int32),
        pltpu.VMEM((K,), jnp.int32),
        pltpu.VMEM((K,), jnp.int32),
        pltpu.VMEM((K,), jnp.int32),
        pltpu.VMEM((K,), jnp.int32),
        pltpu.VMEM((K,), jnp.int32),
        pltpu.VMEM((K, DH), jnp.float32),
        pltpu.VMEM((K, DH), jnp.float32),
        pltpu.VMEM((K, DH), jnp.float32),
        pltpu.VMEM((K, DH), jnp.float32),
        pltpu.VMEM((K, DW), jnp.float32),
        pltpu.VMEM((K, DW), jnp.float32),
        pltpu.VMEM_SHARED((NPAD, DW), jnp.float32),
        pltpu.SemaphoreType.DMA,
        pltpu.SemaphoreType.DMA,
        pltpu.SemaphoreType.DMA,
        pltpu.SemaphoreType.DMA,
        pltpu.SemaphoreType.DMA,
        pltpu.SemaphoreType.DMA,
    ],
    compiler_params=pltpu.CompilerParams(use_tc_tiling_on_sc=False,
                                         needs_layout_passes=False),
)
def _edge(A_hbm, B_hbm, src_hbm, dst_hbm, S_out,
          di0, si0, dc0, di1, si1, dc1, ab0, bb0, ab1, bb1, rr0, rr1,
          S_sh, sg0, sg1, mi0, mi1, md0, md1):
    cid = lax.axis_index("c")
    sid = lax.axis_index("s")
    wid = sid * NC + cid

    zv = jnp.zeros((16,), jnp.float32)
    ov = jnp.ones((16,), jnp.float32)

    SETS = ((di0, si0, dc0, ab0, bb0, rr0, sg0, mi0, md0),
            (di1, si1, dc1, ab1, bb1, rr1, sg1, mi1, md1))

    def zrow(i, _):
        for t in range(DW // 16):
            rr0[i, pl.ds(t * 16, 16)] = zv
        return 0
    lax.fori_loop(0, K, zrow, 0)

    def zs(t, _):
        base = sid * RPT + t * K
        pltpu.sync_copy(rr0, S_sh.at[pl.ds(base, K)])
        return 0
    lax.fori_loop(0, RPT // K, zs, 0)
    plsc.subcore_barrier()

    def orow(i, _):
        rr0[i, pl.ds(DH, 16)] = ov
        rr1[i, pl.ds(DH, 16)] = ov
        return 0
    lax.fori_loop(0, K, orow, 0)

    def idx_load(j, p):
        di, si = SETS[p][0], SETS[p][1]
        mi = SETS[p][7]
        pltpu.async_copy(dst_hbm.at[wid, j], di, mi)
        pltpu.async_copy(src_hbm.at[wid, j], si, mi)

    def idx_wait(p):
        di, si = SETS[p][0], SETS[p][1]
        mi = SETS[p][7]
        pltpu.make_async_copy(dst_hbm.at[wid, 0], di, mi).wait()
        pltpu.make_async_copy(src_hbm.at[wid, 0], si, mi).wait()

    def gather_launch(p):
        di, si, _, ab, bb, _, sg = SETS[p][:7]
        pltpu.async_copy(A_hbm.at[di], ab, sg)
        pltpu.async_copy(B_hbm.at[si], bb, sg)

    def gather_wait(p):
        di, si, _, ab, bb, _, sg = SETS[p][:7]
        pltpu.make_async_copy(A_hbm.at[di], ab, sg).wait()
        pltpu.make_async_copy(B_hbm.at[si], bb, sg).wait()

    def phase(j, p, first=False, launch=True):
        q = 1 - p
        di, si, dc, ab, bb, rr, sg, mi, md = SETS[p]
        dc_q, md_q = SETS[q][2], SETS[q][8]
        gather_wait(p)
        def row(i, _):
            for r in range(2):
                ii = i * 2 + r
                for t in range(DH // 16):
                    sl = pl.ds(t * 16, 16)
                    rr[ii, sl] = jnp.maximum(ab[ii, sl] + bb[ii, sl], 0.0)
            return 0
        lax.fori_loop(0, K // 2, row, 0)
        pltpu.make_async_copy(dst_hbm.at[wid, 0], dc, md).wait()
        pltpu.sync_copy(rr, S_sh.at[dc], add=True)
        if launch:
            pltpu.async_copy(dst_hbm.at[wid, jnp.minimum(j + 1, NCH - 1)],
                             dc_q, md_q)
            idx_wait(q)
            gather_launch(q)
            idx_load(jnp.minimum(j + 2, NCH - 1), p)

    idx_load(0, 0)
    idx_load(1, 1)
    pltpu.async_copy(dst_hbm.at[wid, 0], dc0, md0)
    idx_wait(0)
    gather_launch(0)
    phase(0, 0, first=True)

    def loop_body(t, _):
        phase(2 * t + 1, 1)
        phase(2 * t + 2, 0)
        return 0
    lax.fori_loop(0, (NCH - 2) // 2, loop_body, 0)

    phase(NCH - 1, 1, launch=False)
    idx_wait(0)
    plsc.subcore_barrier()

    def wb(t, _):
        base = sid * RPT + t * K
        pltpu.sync_copy(S_sh.at[pl.ds(base, K)], rr0)
        pltpu.sync_copy(rr0, S_out.at[cid, pl.ds(base, K)])
        return 0
    lax.fori_loop(0, RPT // K, wb, 0)


def _fin_body(S_ref, W2_ref, b2_ref, o_ref):
    s = S_ref[0, :, :DH] + S_ref[1, :, :DH]
    deg16 = S_ref[0, :, DH:] + S_ref[1, :, DH:]
    sel = (lax.broadcasted_iota(jnp.int32, (16, 1), 0) == 0).astype(jnp.float32)
    degb2 = jnp.dot(deg16, sel * b2_ref[0, :][None, :],
                    preferred_element_type=jnp.float32)
    o_ref[...] = (jnp.dot(s, W2_ref[...], preferred_element_type=jnp.float32)
                  + degb2)


_fin = pl.pallas_call(
    _fin_body,
    grid=(NB,),
    in_specs=[
        pl.BlockSpec((NC, BN, DW), lambda i: (0, i, 0)),
        pl.BlockSpec((DH, DH), lambda i: (0, 0)),
        pl.BlockSpec((1, DH), lambda i: (0, 0)),
    ],
    out_specs=pl.BlockSpec((BN, DH), lambda i: (i, 0)),
    out_shape=jax.ShapeDtypeStruct((N, DH), jnp.float32),
)


def kernel(x, scalars, edge_index, batch, W1, b1, W2, b2):
    srcr = edge_index[0].reshape(NW, NCH, K)
    dstr = edge_index[1].reshape(NW, NCH, K)
    bt3 = batch.reshape(NB, 1, BN)
    A, B = _pre(x, bt3, scalars, W1, b1.reshape(1, DH))
    S = _edge(A, B, srcr, dstr)
    return _fin(S, W2, b2.reshape(1, DH))


# speedup vs baseline: 1.9572x; 1.9572x over previous
import functools

import jax
import jax.numpy as jnp
import numpy as np
from jax import lax
from jax.experimental import pallas as pl
from jax.experimental.pallas import tpu as pltpu
from jax.experimental.pallas import tpu_sc as plsc

N = 10000
E = 320000
DF = 128
NS = 8
NG = 16
DH = 128

NB = 10
BN = N // NB

NC = 2
NSUB = 16
NW = NC * NSUB
EPW = E // NW
K = 40
NCH = EPW // K
NPAD = 10240
RPT = NPAD // NSUB


def _pre_body(x_ref, bt_ref, sc_ref, W1_ref, b1_ref, A_ref, B_ref):
    xb = x_ref[...]
    bt = bt_ref[0, 0, :]
    oh = (bt[:, None] == lax.broadcasted_iota(jnp.int32, (1, NG), 1)
          ).astype(jnp.float32)
    s = jnp.dot(oh, sc_ref[...], preferred_element_type=jnp.float32)
    w1ax = W1_ref[0:DF, :]
    w1as = W1_ref[DF:DF + NS, :]
    w1bx = W1_ref[DF + NS:2 * DF + NS, :]
    w1bs = W1_ref[2 * DF + NS:, :]
    b1 = b1_ref[0, :]
    A_ref[...] = (jnp.dot(xb, w1ax - w1bx, preferred_element_type=jnp.float32)
                  + jnp.dot(s, w1as - w1bs, preferred_element_type=jnp.float32)
                  + b1[None, :])
    B_ref[...] = (jnp.dot(xb, w1bx, preferred_element_type=jnp.float32)
                  + jnp.dot(s, w1bs, preferred_element_type=jnp.float32))


_pre = pl.pallas_call(
    _pre_body,
    grid=(NB,),
    in_specs=[
        pl.BlockSpec((BN, DF), lambda i: (i, 0)),
        pl.BlockSpec((1, 1, BN), lambda i: (i, 0, 0)),
        pl.BlockSpec((NG, NS), lambda i: (0, 0)),
        pl.BlockSpec((2 * (DF + NS), DH), lambda i: (0, 0)),
        pl.BlockSpec((1, DH), lambda i: (0, 0)),
    ],
    out_specs=[pl.BlockSpec((BN, DH), lambda i: (i, 0)),
               pl.BlockSpec((BN, DH), lambda i: (i, 0))],
    out_shape=[jax.ShapeDtypeStruct((N, DH), jnp.float32),
               jax.ShapeDtypeStruct((N, DH), jnp.float32)],
)


_mesh = plsc.VectorSubcoreMesh(core_axis_name="c", subcore_axis_name="s")


@functools.partial(
    pl.kernel,
    out_type=(jax.ShapeDtypeStruct((NC, NPAD, DH), jnp.float32),
              jax.ShapeDtypeStruct((NC, NPAD, 16), jnp.float32)),
    mesh=_mesh,
    scratch_types=[
        pltpu.VMEM((K,), jnp.int32),
        pltpu.VMEM((K,), jnp.int32),
        pltpu.VMEM((K,), jnp.int32),
        pltpu.VMEM((K,), jnp.int32),
        pltpu.VMEM((K,), jnp.int32),
        pltpu.VMEM((K,), jnp.int32),
        pltpu.VMEM((K, DH), jnp.float32),
        pltpu.VMEM((K, DH), jnp.float32),
        pltpu.VMEM((K, DH), jnp.float32),
        pltpu.VMEM((K, DH), jnp.float32),
        pltpu.VMEM((K, DH), jnp.float32),
        pltpu.VMEM((K, DH), jnp.float32),
        pltpu.VMEM((K, 16), jnp.float32),
        pltpu.VMEM_SHARED((NPAD, DH), jnp.float32),
        pltpu.VMEM_SHARED((NPAD, 16), jnp.float32),
        pltpu.SemaphoreType.DMA,
        pltpu.SemaphoreType.DMA,
        pltpu.SemaphoreType.DMA,
        pltpu.SemaphoreType.DMA,
        pltpu.SemaphoreType.DMA,
        pltpu.SemaphoreType.DMA,
        pltpu.SemaphoreType.DMA,
        pltpu.SemaphoreType.DMA,
    ],
    compiler_params=pltpu.CompilerParams(use_tc_tiling_on_sc=False,
                                         needs_layout_passes=False),
)
def _edge(A_hbm, B_hbm, src_hbm, dst_hbm, S_out, Dg_out,
          di0, si0, dc0, di1, si1, dc1, ab0, bb0, ab1, bb1, rr0, rr1,
          ones_v, S_sh, D_sh, sg0, sg1, ss0, ss1, mi0, mi1, md0, md1):
    cid = lax.axis_index("c")
    sid = lax.axis_index("s")
    wid = sid * NC + cid

    zv = jnp.zeros((16,), jnp.float32)
    ov = jnp.ones((16,), jnp.float32)

    SETS = ((di0, si0, dc0, ab0, bb0, rr0, sg0, ss0, mi0, md0),
            (di1, si1, dc1, ab1, bb1, rr1, sg1, ss1, mi1, md1))

    def zrow(i, _):
        for t in range(DH // 16):
            rr0[i, pl.ds(t * 16, 16)] = zv
        ones_v[i, :] = zv
        return 0
    lax.fori_loop(0, K, zrow, 0)

    def zs(t, _):
        base = sid * RPT + t * K
        pltpu.sync_copy(rr0, S_sh.at[pl.ds(base, K)])
        pltpu.sync_copy(ones_v, D_sh.at[pl.ds(base, K)])
        return 0
    lax.fori_loop(0, RPT // K, zs, 0)
    plsc.subcore_barrier()

    def orow(i, _):
        ones_v[i, :] = ov
        return 0
    lax.fori_loop(0, K, orow, 0)

    def idx_load(j, p):
        di, si = SETS[p][0], SETS[p][1]
        mi = SETS[p][8]
        pltpu.async_copy(dst_hbm.at[wid, j], di, mi)
        pltpu.async_copy(src_hbm.at[wid, j], si, mi)

    def idx_wait(p):
        di, si = SETS[p][0], SETS[p][1]
        mi = SETS[p][8]
        pltpu.make_async_copy(dst_hbm.at[wid, 0], di, mi).wait()
        pltpu.make_async_copy(src_hbm.at[wid, 0], si, mi).wait()

    def gather_launch(p):
        di, si, _, ab, bb = SETS[p][:5]
        sg = SETS[p][6]
        pltpu.async_copy(A_hbm.at[di], ab, sg)
        pltpu.async_copy(B_hbm.at[si], bb, sg)

    def gather_wait(p):
        di, si, _, ab, bb = SETS[p][:5]
        sg = SETS[p][6]
        pltpu.make_async_copy(A_hbm.at[di], ab, sg).wait()
        pltpu.make_async_copy(B_hbm.at[si], bb, sg).wait()

    def scatter_wait(p):
        dc, rr, ss = SETS[p][2], SETS[p][5], SETS[p][7]
        pltpu.make_async_copy(rr, S_sh.at[dc], ss).wait()
        pltpu.make_async_copy(ones_v, D_sh.at[dc], ss).wait()

    def phase(j, p, drain=True, launch=True):
        q = 1 - p
        di, si, dc, ab, bb, rr, sg, ss, mi, md = SETS[p]
        if drain:
            scatter_wait(p)
        pltpu.async_copy(dst_hbm.at[wid, j], dc, md)
        gather_wait(p)
        def row(i, _):
            for r in range(2):
                ii = i * 2 + r
                for t in range(DH // 16):
                    sl = pl.ds(t * 16, 16)
                    rr[ii, sl] = jnp.maximum(ab[ii, sl] + bb[ii, sl], 0.0)
            return 0
        lax.fori_loop(0, K // 2, row, 0)
        pltpu.make_async_copy(dst_hbm.at[wid, 0], dc, md).wait()
        pltpu.async_copy(rr, S_sh.at[dc], ss, add=True)
        pltpu.async_copy(ones_v, D_sh.at[dc], ss, add=True)
        if launch:
            idx_wait(q)
            gather_launch(q)
            idx_load(jnp.minimum(j + 2, NCH - 1), p)

    idx_load(0, 0)
    idx_load(1, 1)
    idx_wait(0)
    gather_launch(0)
    phase(0, 0, drain=False)
    phase(1, 1, drain=False)

    def loop_body(t, _):
        phase(2 * t + 2, 0)
        phase(2 * t + 3, 1)
        return 0
    lax.fori_loop(0, (NCH - 3) // 2, loop_body, 0)

    phase(NCH - 2, 0)
    phase(NCH - 1, 1, launch=False)
    scatter_wait(0)
    scatter_wait(1)
    idx_wait(0)
    plsc.subcore_barrier()

    def wb(t, _):
        base = sid * RPT + t * K
        pltpu.sync_copy(S_sh.at[pl.ds(base, K)], rr0)
        pltpu.sync_copy(rr0, S_out.at[cid, pl.ds(base, K)])
        pltpu.sync_copy(D_sh.at[pl.ds(base, K)], ones_v)
        pltpu.sync_copy(ones_v, Dg_out.at[cid, pl.ds(base, K)])
        return 0
    lax.fori_loop(0, RPT // K, wb, 0)


def _fin_body(S_ref, Dg_ref, W2_ref, b2_ref, o_ref):
    s = S_ref[0] + S_ref[1]
    deg = Dg_ref[0, :, 0] + Dg_ref[1, :, 0]
    o_ref[...] = (jnp.dot(s, W2_ref[...], preferred_element_type=jnp.float32)
                  + deg[:, None] * b2_ref[0, :][None, :])


_fin = pl.pallas_call(
    _fin_body,
    grid=(NB,),
    in_specs=[
        pl.BlockSpec((NC, BN, DH), lambda i: (0, i, 0)),
        pl.BlockSpec((NC, BN, 16), lambda i: (0, i, 0)),
        pl.BlockSpec((DH, DH), lambda i: (0, 0)),
        pl.BlockSpec((1, DH), lambda i: (0, 0)),
    ],
    out_specs=pl.BlockSpec((BN, DH), lambda i: (i, 0)),
    out_shape=jax.ShapeDtypeStruct((N, DH), jnp.float32),
)


def kernel(x, scalars, edge_index, batch, W1, b1, W2, b2):
    srcr = edge_index[0].reshape(NW, NCH, K)
    dstr = edge_index[1].reshape(NW, NCH, K)
    bt3 = batch.reshape(NB, 1, BN)
    A, B = _pre(x, bt3, scalars, W1, b1.reshape(1, DH))
    S, Dg = _edge(A, B, srcr, dstr)
    return _fin(S, Dg, W2, b2.reshape(1, DH))


# speedup vs baseline: 3.0233x; 1.5447x over previous
import functools

import jax
import jax.numpy as jnp
import numpy as np
from jax import lax
from jax.experimental import pallas as pl
from jax.experimental.pallas import tpu as pltpu
from jax.experimental.pallas import tpu_sc as plsc

N = 10000
E = 320000
DF = 128
NS = 8
NG = 16
DH = 128

NB = 10
BN = N // NB

NC = 2
NSUB = 16
NW = NC * NSUB
EPW = E // NW
K = 40
NCH = EPW // K
NPAD = 10240
RPT = NPAD // NSUB


def _pre_body(x_ref, bt_ref, sc_ref, W1_ref, b1_ref, A_ref, B_ref):
    xb = x_ref[...]
    bt = bt_ref[0, 0, :]
    oh = (bt[:, None] == lax.broadcasted_iota(jnp.int32, (1, NG), 1)
          ).astype(jnp.float32)
    s = jnp.dot(oh, sc_ref[...], preferred_element_type=jnp.float32)
    w1ax = W1_ref[0:DF, :]
    w1as = W1_ref[DF:DF + NS, :]
    w1bx = W1_ref[DF + NS:2 * DF + NS, :]
    w1bs = W1_ref[2 * DF + NS:, :]
    b1 = b1_ref[0, :]
    A_ref[...] = (jnp.dot(xb, w1ax - w1bx, preferred_element_type=jnp.float32)
                  + jnp.dot(s, w1as - w1bs, preferred_element_type=jnp.float32)
                  + b1[None, :])
    B_ref[...] = (jnp.dot(xb, w1bx, preferred_element_type=jnp.float32)
                  + jnp.dot(s, w1bs, preferred_element_type=jnp.float32))


_pre = pl.pallas_call(
    _pre_body,
    grid=(NB,),
    in_specs=[
        pl.BlockSpec((BN, DF), lambda i: (i, 0)),
        pl.BlockSpec((1, 1, BN), lambda i: (i, 0, 0)),
        pl.BlockSpec((NG, NS), lambda i: (0, 0)),
        pl.BlockSpec((2 * (DF + NS), DH), lambda i: (0, 0)),
        pl.BlockSpec((1, DH), lambda i: (0, 0)),
    ],
    out_specs=[pl.BlockSpec((BN, DH), lambda i: (i, 0)),
               pl.BlockSpec((BN, DH), lambda i: (i, 0))],
    out_shape=[jax.ShapeDtypeStruct((N, DH), jnp.float32),
               jax.ShapeDtypeStruct((N, DH), jnp.float32)],
)


_mesh = plsc.VectorSubcoreMesh(core_axis_name="c", subcore_axis_name="s")


@functools.partial(
    pl.kernel,
    out_type=(jax.ShapeDtypeStruct((NC, NPAD, DH), jnp.float32),
              jax.ShapeDtypeStruct((NC, NPAD, 16), jnp.float32)),
    mesh=_mesh,
    scratch_types=[
        pltpu.VMEM((K,), jnp.int32),
        pltpu.VMEM((K,), jnp.int32),
        pltpu.VMEM((K,), jnp.int32),
        pltpu.VMEM((K,), jnp.int32),
        pltpu.VMEM((K,), jnp.int32),
        pltpu.VMEM((K,), jnp.int32),
        pltpu.VMEM((K, DH), jnp.float32),
        pltpu.VMEM((K, DH), jnp.float32),
        pltpu.VMEM((K, DH), jnp.float32),
        pltpu.VMEM((K, DH), jnp.float32),
        pltpu.VMEM((K, DH), jnp.float32),
        pltpu.VMEM((K, DH), jnp.float32),
        pltpu.VMEM((K, 16), jnp.float32),
        pltpu.VMEM_SHARED((NPAD, DH), jnp.float32),
        pltpu.VMEM_SHARED((NPAD, 16), jnp.float32),
        pltpu.SemaphoreType.DMA,
        pltpu.SemaphoreType.DMA,
        pltpu.SemaphoreType.DMA,
        pltpu.SemaphoreType.DMA,
        pltpu.SemaphoreType.DMA,
        pltpu.SemaphoreType.DMA,
        pltpu.SemaphoreType.DMA,
        pltpu.SemaphoreType.DMA,
    ],
    compiler_params=pltpu.CompilerParams(use_tc_tiling_on_sc=False,
                                         needs_layout_passes=False),
)
def _edge(A_hbm, B_hbm, src_hbm, dst_hbm, S_out, Dg_out,
          di0, si0, dc0, di1, si1, dc1, ab0, bb0, ab1, bb1, rr0, rr1,
          ones_v, S_sh, D_sh, sg0, sg1, ss0, ss1, mi0, mi1, md0, md1):
    cid = lax.axis_index("c")
    sid = lax.axis_index("s")
    wid = sid * NC + cid

    zv = jnp.zeros((16,), jnp.float32)
    ov = jnp.ones((16,), jnp.float32)

    SETS = ((di0, si0, dc0, ab0, bb0, rr0, sg0, ss0, mi0, md0),
            (di1, si1, dc1, ab1, bb1, rr1, sg1, ss1, mi1, md1))

    def zrow(i, _):
        for t in range(DH // 16):
            rr0[i, pl.ds(t * 16, 16)] = zv
        ones_v[i, :] = zv
        return 0
    lax.fori_loop(0, K, zrow, 0)

    def zs(t, _):
        base = sid * RPT + t * K
        pltpu.sync_copy(rr0, S_sh.at[pl.ds(base, K)])
        pltpu.sync_copy(ones_v, D_sh.at[pl.ds(base, K)])
        return 0
    lax.fori_loop(0, RPT // K, zs, 0)
    plsc.subcore_barrier()

    def orow(i, _):
        ones_v[i, :] = ov
        return 0
    lax.fori_loop(0, K, orow, 0)

    def idx_load(j, p):
        di, si = SETS[p][0], SETS[p][1]
        mi = SETS[p][8]
        pltpu.async_copy(dst_hbm.at[wid, j], di, mi)
        pltpu.async_copy(src_hbm.at[wid, j], si, mi)

    def idx_wait(p):
        di, si = SETS[p][0], SETS[p][1]
        mi = SETS[p][8]
        pltpu.make_async_copy(dst_hbm.at[wid, 0], di, mi).wait()
        pltpu.make_async_copy(src_hbm.at[wid, 0], si, mi).wait()

    def gather_launch(p):
        di, si, _, ab, bb = SETS[p][:5]
        sg = SETS[p][6]
        pltpu.async_copy(A_hbm.at[di], ab, sg)
        pltpu.async_copy(B_hbm.at[si], bb, sg)

    def gather_wait(p):
        di, si, _, ab, bb = SETS[p][:5]
        sg = SETS[p][6]
        pltpu.make_async_copy(A_hbm.at[di], ab, sg).wait()
        pltpu.make_async_copy(B_hbm.at[si], bb, sg).wait()

    def scatter_wait(p):
        dc, rr, ss = SETS[p][2], SETS[p][5], SETS[p][7]
        pltpu.make_async_copy(rr, S_sh.at[dc], ss).wait()
        pltpu.make_async_copy(ones_v, D_sh.at[dc], ss).wait()

    def phase(j, p, drain=True, launch=True):
        di, si, dc, ab, bb, rr, sg, ss, mi, md = SETS[p]
        if drain:
            scatter_wait(p)
        pltpu.async_copy(dst_hbm.at[wid, j], dc, md)
        gather_wait(p)
        if launch:
            idx_load(j + 2, p)
        def row(i, _):
            for r in range(2):
                ii = i * 2 + r
                for t in range(DH // 16):
                    sl = pl.ds(t * 16, 16)
                    rr[ii, sl] = jnp.maximum(ab[ii, sl] + bb[ii, sl], 0.0)
            return 0
        lax.fori_loop(0, K // 2, row, 0)
        pltpu.make_async_copy(dst_hbm.at[wid, 0], dc, md).wait()
        pltpu.async_copy(rr, S_sh.at[dc], ss, add=True)
        pltpu.async_copy(ones_v, D_sh.at[dc], ss, add=True)
        if launch:
            idx_wait(p)
            gather_launch(p)

    idx_load(0, 0)
    idx_load(1, 1)
    idx_wait(0)
    gather_launch(0)
    idx_wait(1)
    gather_launch(1)
    phase(0, 0, drain=False)
    phase(1, 1, drain=False)

    def loop_body(t, _):
        phase(2 * t + 2, 0)
        phase(2 * t + 3, 1)
        return 0
    lax.fori_loop(0, (NCH - 4) // 2, loop_body, 0)

    phase(NCH - 2, 0, launch=False)
    phase(NCH - 1, 1, launch=False)
    scatter_wait(0)
    scatter_wait(1)
    plsc.subcore_barrier()

    def wb(t, _):
        base = sid * RPT + t * K
        pltpu.sync_copy(S_sh.at[pl.ds(base, K)], rr0)
        pltpu.sync_copy(rr0, S_out.at[cid, pl.ds(base, K)])
        pltpu.sync_copy(D_sh.at[pl.ds(base, K)], ones_v)
        pltpu.sync_copy(ones_v, Dg_out.at[cid, pl.ds(base, K)])
        return 0
    lax.fori_loop(0, RPT // K, wb, 0)


def _fin_body(S_ref, Dg_ref, W2_ref, b2_ref, o_ref):
    s = S_ref[0] + S_ref[1]
    deg = Dg_ref[0, :, 0] + Dg_ref[1, :, 0]
    o_ref[...] = (jnp.dot(s, W2_ref[...], preferred_element_type=jnp.float32)
                  + deg[:, None] * b2_ref[0, :][None, :])


_fin = pl.pallas_call(
    _fin_body,
    grid=(NB,),
    in_specs=[
        pl.BlockSpec((NC, BN, DH), lambda i: (0, i, 0)),
        pl.BlockSpec((NC, BN, 16), lambda i: (0, i, 0)),
        pl.BlockSpec((DH, DH), lambda i: (0, 0)),
        pl.BlockSpec((1, DH), lambda i: (0, 0)),
    ],
    out_specs=pl.BlockSpec((BN, DH), lambda i: (i, 0)),
    out_shape=jax.ShapeDtypeStruct((N, DH), jnp.float32),
)


def kernel(x, scalars, edge_index, batch, W1, b1, W2, b2):
    srcr = edge_index[0].reshape(NW, NCH, K)
    dstr = edge_index[1].reshape(NW, NCH, K)
    bt3 = batch.reshape(NB, 1, BN)
    A, B = _pre(x, bt3, scalars, W1, b1.reshape(1, DH))
    S, Dg = _edge(A, B, srcr, dstr)
    return _fin(S, Dg, W2, b2.reshape(1, DH))


# speedup vs baseline: 3.0891x; 1.0218x over previous
import functools

import jax
import jax.numpy as jnp
import numpy as np
from jax import lax
from jax.experimental import pallas as pl
from jax.experimental.pallas import tpu as pltpu
from jax.experimental.pallas import tpu_sc as plsc

N = 10000
E = 320000
DF = 128
NS = 8
NG = 16
DH = 128

NB = 5
BN = N // NB

NC = 2
NSUB = 16
NW = NC * NSUB
EPW = E // NW
K = 40
NCH = EPW // K
NPAD = 10240
RPT = NPAD // NSUB


def _pre_body(x_ref, bt_ref, sc_ref, W1_ref, b1_ref, A_ref, B_ref):
    xb = x_ref[...]
    bt = bt_ref[0, 0, :]
    oh = (bt[:, None] == lax.broadcasted_iota(jnp.int32, (1, NG), 1)
          ).astype(jnp.float32)
    s = jnp.dot(oh, sc_ref[...], preferred_element_type=jnp.float32)
    w1ax = W1_ref[0:DF, :]
    w1as = W1_ref[DF:DF + NS, :]
    w1bx = W1_ref[DF + NS:2 * DF + NS, :]
    w1bs = W1_ref[2 * DF + NS:, :]
    b1 = b1_ref[0, :]
    A_ref[...] = (jnp.dot(xb, w1ax - w1bx, preferred_element_type=jnp.float32)
                  + jnp.dot(s, w1as - w1bs, preferred_element_type=jnp.float32)
                  + b1[None, :])
    B_ref[...] = (jnp.dot(xb, w1bx, preferred_element_type=jnp.float32)
                  + jnp.dot(s, w1bs, preferred_element_type=jnp.float32))


_pre = pl.pallas_call(
    _pre_body,
    grid=(NB,),
    in_specs=[
        pl.BlockSpec((BN, DF), lambda i: (i, 0)),
        pl.BlockSpec((1, 1, BN), lambda i: (i, 0, 0)),
        pl.BlockSpec((NG, NS), lambda i: (0, 0)),
        pl.BlockSpec((2 * (DF + NS), DH), lambda i: (0, 0)),
        pl.BlockSpec((1, DH), lambda i: (0, 0)),
    ],
    out_specs=[pl.BlockSpec((BN, DH), lambda i: (i, 0)),
               pl.BlockSpec((BN, DH), lambda i: (i, 0))],
    out_shape=[jax.ShapeDtypeStruct((N, DH), jnp.float32),
               jax.ShapeDtypeStruct((N, DH), jnp.float32)],
)


_mesh = plsc.VectorSubcoreMesh(core_axis_name="c", subcore_axis_name="s")


@functools.partial(
    pl.kernel,
    out_type=(jax.ShapeDtypeStruct((NC, NPAD, DH), jnp.float32),
              jax.ShapeDtypeStruct((NC, NPAD, 16), jnp.float32)),
    mesh=_mesh,
    scratch_types=[
        pltpu.VMEM((K,), jnp.int32),
        pltpu.VMEM((K,), jnp.int32),
        pltpu.VMEM((K,), jnp.int32),
        pltpu.VMEM((K,), jnp.int32),
        pltpu.VMEM((K,), jnp.int32),
        pltpu.VMEM((K,), jnp.int32),
        pltpu.VMEM((K, DH), jnp.float32),
        pltpu.VMEM((K, DH), jnp.float32),
        pltpu.VMEM((K, DH), jnp.float32),
        pltpu.VMEM((K, DH), jnp.float32),
        pltpu.VMEM((K, DH), jnp.float32),
        pltpu.VMEM((K, DH), jnp.float32),
        pltpu.VMEM((K, 16), jnp.float32),
        pltpu.VMEM_SHARED((NPAD, DH), jnp.float32),
        pltpu.VMEM_SHARED((NPAD, 16), jnp.float32),
        pltpu.SemaphoreType.DMA,
        pltpu.SemaphoreType.DMA,
        pltpu.SemaphoreType.DMA,
        pltpu.SemaphoreType.DMA,
        pltpu.SemaphoreType.DMA,
        pltpu.SemaphoreType.DMA,
        pltpu.SemaphoreType.DMA,
        pltpu.SemaphoreType.DMA,
    ],
    compiler_params=pltpu.CompilerParams(use_tc_tiling_on_sc=False,
                                         needs_layout_passes=False),
)
def _edge(A_hbm, B_hbm, src_hbm, dst_hbm, S_out, Dg_out,
          di0, si0, dc0, di1, si1, dc1, ab0, bb0, ab1, bb1, rr0, rr1,
          ones_v, S_sh, D_sh, sg0, sg1, ss0, ss1, mi0, mi1, md0, md1):
    cid = lax.axis_index("c")
    sid = lax.axis_index("s")
    wid = sid * NC + cid

    zv = jnp.zeros((16,), jnp.float32)
    ov = jnp.ones((16,), jnp.float32)

    SETS = ((di0, si0, dc0, ab0, bb0, rr0, sg0, ss0, mi0, md0),
            (di1, si1, dc1, ab1, bb1, rr1, sg1, ss1, mi1, md1))

    def zrow(i, _):
        for t in range(DH // 16):
            rr0[i, pl.ds(t * 16, 16)] = zv
        ones_v[i, :] = zv
        return 0
    lax.fori_loop(0, K, zrow, 0)

    def zs(t, _):
        base = sid * RPT + t * K
        pltpu.sync_copy(rr0, S_sh.at[pl.ds(base, K)])
        pltpu.sync_copy(ones_v, D_sh.at[pl.ds(base, K)])
        return 0
    lax.fori_loop(0, RPT // K, zs, 0)
    plsc.subcore_barrier()

    def orow(i, _):
        ones_v[i, :] = ov
        return 0
    lax.fori_loop(0, K, orow, 0)

    def idx_load(j, p):
        di, si = SETS[p][0], SETS[p][1]
        mi = SETS[p][8]
        pltpu.async_copy(dst_hbm.at[wid, j], di, mi)
        pltpu.async_copy(src_hbm.at[wid, j], si, mi)

    def idx_wait(p):
        di, si = SETS[p][0], SETS[p][1]
        mi = SETS[p][8]
        pltpu.make_async_copy(dst_hbm.at[wid, 0], di, mi).wait()
        pltpu.make_async_copy(src_hbm.at[wid, 0], si, mi).wait()

    def gather_launch(p):
        di, si, _, ab, bb = SETS[p][:5]
        sg = SETS[p][6]
        pltpu.async_copy(A_hbm.at[di], ab, sg)
        pltpu.async_copy(B_hbm.at[si], bb, sg)

    def gather_wait(p):
        di, si, _, ab, bb = SETS[p][:5]
        sg = SETS[p][6]
        pltpu.make_async_copy(A_hbm.at[di], ab, sg).wait()
        pltpu.make_async_copy(B_hbm.at[si], bb, sg).wait()

    def scatter_wait(p):
        dc, rr, ss = SETS[p][2], SETS[p][5], SETS[p][7]
        pltpu.make_async_copy(rr, S_sh.at[dc], ss).wait()
        pltpu.make_async_copy(ones_v, D_sh.at[dc], ss).wait()

    def phase(j, p, drain=True, launch=True):
        di, si, dc, ab, bb, rr, sg, ss, mi, md = SETS[p]
        if drain:
            scatter_wait(p)
        pltpu.async_copy(dst_hbm.at[wid, j], dc, md)
        gather_wait(p)
        if launch:
            idx_load(j + 2, p)
        def row(i, _):
            for r in range(2):
                ii = i * 2 + r
                for t in range(DH // 16):
                    sl = pl.ds(t * 16, 16)
                    rr[ii, sl] = jnp.maximum(ab[ii, sl] + bb[ii, sl], 0.0)
            return 0
        lax.fori_loop(0, K // 2, row, 0)
        if launch:
            idx_wait(p)
            gather_launch(p)
        pltpu.make_async_copy(dst_hbm.at[wid, 0], dc, md).wait()
        pltpu.async_copy(rr, S_sh.at[dc], ss, add=True)
        pltpu.async_copy(ones_v, D_sh.at[dc], ss, add=True)

    idx_load(0, 0)
    idx_load(1, 1)
    idx_wait(0)
    gather_launch(0)
    idx_wait(1)
    gather_launch(1)
    phase(0, 0, drain=False)
    phase(1, 1, drain=False)

    def loop_body(t, _):
        phase(2 * t + 2, 0)
        phase(2 * t + 3, 1)
        return 0
    lax.fori_loop(0, (NCH - 4) // 2, loop_body, 0)

    phase(NCH - 2, 0, launch=False)
    phase(NCH - 1, 1, launch=False)
    scatter_wait(0)
    scatter_wait(1)
    plsc.subcore_barrier()

    def wb(t, _):
        base = sid * RPT + t * K
        pltpu.sync_copy(S_sh.at[pl.ds(base, K)], rr0)
        pltpu.sync_copy(rr0, S_out.at[cid, pl.ds(base, K)])
        pltpu.sync_copy(D_sh.at[pl.ds(base, K)], ones_v)
        pltpu.sync_copy(ones_v, Dg_out.at[cid, pl.ds(base, K)])
        return 0
    lax.fori_loop(0, RPT // K, wb, 0)


def _fin_body(S_ref, Dg_ref, W2_ref, b2_ref, o_ref):
    s = S_ref[0] + S_ref[1]
    deg = Dg_ref[0, :, 0] + Dg_ref[1, :, 0]
    o_ref[...] = (jnp.dot(s, W2_ref[...], preferred_element_type=jnp.float32)
                  + deg[:, None] * b2_ref[0, :][None, :])


_fin = pl.pallas_call(
    _fin_body,
    grid=(NB,),
    in_specs=[
        pl.BlockSpec((NC, BN, DH), lambda i: (0, i, 0)),
        pl.BlockSpec((NC, BN, 16), lambda i: (0, i, 0)),
        pl.BlockSpec((DH, DH), lambda i: (0, 0)),
        pl.BlockSpec((1, DH), lambda i: (0, 0)),
    ],
    out_specs=pl.BlockSpec((BN, DH), lambda i: (i, 0)),
    out_shape=jax.ShapeDtypeStruct((N, DH), jnp.float32),
)


def kernel(x, scalars, edge_index, batch, W1, b1, W2, b2):
    srcr = edge_index[0].reshape(NW, NCH, K)
    dstr = edge_index[1].reshape(NW, NCH, K)
    bt3 = batch.reshape(NB, 1, BN)
    A, B = _pre(x, bt3, scalars, W1, b1.reshape(1, DH))
    S, Dg = _edge(A, B, srcr, dstr)
    return _fin(S, Dg, W2, b2.reshape(1, DH))
